# Initial kernel scaffold; baseline (speedup 1.0000x reference)
import jax
import jax.numpy as jnp
from jax.experimental import pallas as pl


def kernel(x, edge_index, W1, b1, W2, b2):
    raise NotImplementedError("write your pallas kernel here")


# speedup vs baseline: 13.9312x; 13.9312x over previous
import functools
import jax
import jax.numpy as jnp
from jax import lax
from jax.experimental import pallas as pl
from jax.experimental.pallas import tpu as pltpu
from jax.experimental.pallas import tpu_sc as plsc

N = 10000
D_IN = 256
D_OUT = 256
E = 160000
NP = 10240
EP = 163840
NSLAB = 4
DSLAB = 128
NTILE = 16
NCORE = 2
ROWS_PER_TILE = NP // NTILE
CHUNKS = EP // (NTILE * NCORE) // 128
SCALING_FACTOR = 1.8


@functools.cache
def _sc_kernels():
    mesh = plsc.VectorSubcoreMesh(core_axis_name="c", subcore_axis_name="s")

    NPASS = 4
    HROWS = NP // NPASS
    HPAD = 32
    DRT = HROWS // NTILE

    @functools.partial(
        pl.kernel,
        out_type=jax.ShapeDtypeStruct((NCORE, NP, 128), jnp.float32),
        mesh=mesh,
        scratch_types=[
            pltpu.VMEM((CHUNKS, 128), jnp.int32),
            pltpu.VMEM((CHUNKS, 128), jnp.int32),
            pltpu.VMEM((128, 128), jnp.float32),
            pltpu.VMEM((CHUNKS, 128), jnp.float32),
            pltpu.VMEM((CHUNKS, 128), jnp.float32),
            pltpu.VMEM_SHARED((HROWS + HPAD, 128), jnp.float32),
        ],
    )
    def deg_kernel(dst_hbm, out_hbm, idx_v, idxr_v, ones_v, zbuf, dbuf, acc):
        c = lax.axis_index("c")
        s = lax.axis_index("s")
        blk = c * NTILE + s

        zero16 = jnp.zeros((16,), jnp.float32)
        one16 = jnp.full((16,), 1.0, jnp.float32)

        def _fill_ones(i, _):
            for k in range(8):
                ones_v[i, pl.ds(k * 16, 16)] = one16
            return 0

        lax.fori_loop(0, 128, _fill_ones, 0)

        def _fill_zero(i, _):
            for k in range(8):
                zbuf[i, pl.ds(k * 16, 16)] = zero16
            return 0

        lax.fori_loop(0, CHUNKS, _fill_zero, 0)

        pltpu.sync_copy(dst_hbm.at[blk], idx_v)

        for p in range(NPASS):
            for k in range(DRT // CHUNKS):
                pltpu.sync_copy(
                    zbuf, acc.at[pl.ds(s * DRT + k * CHUNKS, CHUNKS)]
                )
            plsc.subcore_barrier()

            def _remap(j, _):
                for k in range(8):
                    idx16 = idx_v[j, pl.ds(k * 16, 16)]
                    local = idx16 - p * HROWS
                    ok = (local >= 0) & (local < HROWS)
                    junk = HROWS + lax.bitwise_and(idx16, HPAD - 1)
                    idxr_v[j, pl.ds(k * 16, 16)] = jnp.where(ok, local, junk)
                return 0

            lax.fori_loop(0, CHUNKS, _remap, 0)
            for j in range(CHUNKS):
                pltpu.sync_copy(ones_v, acc.at[idxr_v.at[j]], add=True)
            plsc.subcore_barrier()
            for k in range(DRT // CHUNKS):
                pltpu.sync_copy(
                    acc.at[pl.ds(s * DRT + k * CHUNKS, CHUNKS)], dbuf
                )
                pltpu.sync_copy(
                    dbuf,
                    out_hbm.at[
                        c,
                        pl.ds(p * HROWS + s * DRT + k * CHUNKS, CHUNKS),
                    ],
                )
            plsc.subcore_barrier()

    @functools.partial(
        pl.kernel,
        out_type=jax.ShapeDtypeStruct((NSLAB * NP, DSLAB), jnp.float32),
        mesh=mesh,
        scratch_types=[
            pltpu.VMEM((CHUNKS, 128), jnp.int32),
            pltpu.VMEM((CHUNKS, 128), jnp.int32),
            pltpu.VMEM((2, 128, DSLAB), jnp.float32),
            pltpu.VMEM((CHUNKS, DSLAB), jnp.float32),
            pltpu.VMEM_SHARED((NP, DSLAB), jnp.float32),
            pltpu.SemaphoreType.DMA,
            pltpu.SemaphoreType.DMA,
        ],
    )
    def prop_kernel(g_hbm, srcoff_hbm, dst_hbm, out_hbm, src_v, dst_v, rb,
                    zbuf, acc, sem0, sem1):
        c = lax.axis_index("c")
        s = lax.axis_index("s")
        sems = (sem0, sem1)

        zero16 = jnp.zeros((16,), jnp.float32)

        def _fill_zero(i, _):
            for k in range(DSLAB // 16):
                zbuf[i, pl.ds(k * 16, 16)] = zero16
            return 0

        lax.fori_loop(0, CHUNKS, _fill_zero, 0)

        for p in range(NSLAB // NCORE):
            slab = c * (NSLAB // NCORE) + p
            for k in range(ROWS_PER_TILE // CHUNKS):
                pltpu.sync_copy(
                    zbuf,
                    acc.at[pl.ds(s * ROWS_PER_TILE + k * CHUNKS, CHUNKS)],
                )
            plsc.subcore_barrier()
            for ebi in range(2):
                pltpu.sync_copy(dst_hbm.at[ebi * NTILE + s], dst_v)
                pltpu.sync_copy(
                    srcoff_hbm.at[
                        slab * (NTILE * NCORE) + ebi * NTILE + s
                    ],
                    src_v,
                )
                handles = [
                    pltpu.async_copy(g_hbm.at[src_v.at[0]], rb.at[0], sems[0])
                ]
                for j in range(CHUNKS):
                    if j + 1 < CHUNKS:
                        handles.append(
                            pltpu.async_copy(
                                g_hbm.at[src_v.at[j + 1]],
                                rb.at[(j + 1) % 2],
                                sems[(j + 1) % 2],
                            )
                        )
                    handles[j].wait()
                    pltpu.sync_copy(
                        rb.at[j % 2], acc.at[dst_v.at[j]], add=True
                    )
            plsc.subcore_barrier()
            for k in range(ROWS_PER_TILE // 128):
                pltpu.sync_copy(
                    acc.at[pl.ds(s * ROWS_PER_TILE + k * 128, 128)],
                    rb.at[k % 2],
                )
                pltpu.sync_copy(
                    rb.at[k % 2],
                    out_hbm.at[
                        pl.ds(slab * NP + s * ROWS_PER_TILE + k * 128, 128)
                    ],
                )
            plsc.subcore_barrier()

    return deg_kernel, prop_kernel


_BN_A = 512


def _reduce_deg(h0_blk, h1_blk):
    return (
        jnp.sum(h0_blk, axis=1, keepdims=True)
        + jnp.sum(h1_blk, axis=1, keepdims=True)
    ) * (1.0 / 128.0) + 1.0


def _fwd_body(x_ref, w_ref, b_ref, h0_ref, h1_ref, out_ref, dinv_ref):
    xb = x_ref[...]
    y = jnp.dot(xb, w_ref[...], preferred_element_type=jnp.float32) + b_ref[...]
    y1 = y[:, :D_OUT]
    y2 = y[:, D_OUT:]
    nrm = jnp.sqrt(jnp.sum(y2 * y2, axis=1, keepdims=True))
    h = y2 * (SCALING_FACTOR / jnp.maximum(nrm, 1e-12))
    dinv = lax.rsqrt(_reduce_deg(h0_ref[...], h1_ref[...]))
    dinv_ref[...] = dinv
    i = pl.program_id(0)
    rows = i * _BN_A + lax.broadcasted_iota(jnp.int32, (_BN_A, 1), 0)
    mask = rows < N
    g1 = jnp.where(mask, dinv * y1, 0.0)
    gh = jnp.where(mask, dinv * h, 0.0)
    out_ref[0] = g1[:, :DSLAB]
    out_ref[1] = g1[:, DSLAB:]
    out_ref[2] = gh[:, :DSLAB]
    out_ref[3] = gh[:, DSLAB:]


def _make_fwd_call(interpret=False):
    return pl.pallas_call(
        _fwd_body,
        grid=(NP // _BN_A,),
        in_specs=[
            pl.BlockSpec((_BN_A, D_IN), lambda i: (i, 0)),
            pl.BlockSpec((D_IN, 2 * D_OUT), lambda i: (0, 0)),
            pl.BlockSpec((1, 2 * D_OUT), lambda i: (0, 0)),
            pl.BlockSpec((_BN_A, 128), lambda i: (i, 0)),
            pl.BlockSpec((_BN_A, 128), lambda i: (i, 0)),
        ],
        out_specs=[
            pl.BlockSpec((NSLAB, _BN_A, DSLAB), lambda i: (0, i, 0)),
            pl.BlockSpec((_BN_A, 1), lambda i: (i, 0)),
        ],
        out_shape=[
            jax.ShapeDtypeStruct((NSLAB, NP, DSLAB), jnp.float32),
            jax.ShapeDtypeStruct((NP, 1), jnp.float32),
        ],
        interpret=interpret,
    )


_BN_B = 400


def _final_body(s_ref, g_ref, dinv_ref, x_out, h_out):
    dinv = dinv_ref[...]
    o0 = dinv * (s_ref[0] + g_ref[0])
    o1 = dinv * (s_ref[1] + g_ref[1])
    o2 = dinv * (s_ref[2] + g_ref[2])
    o3 = dinv * (s_ref[3] + g_ref[3])
    x_out[...] = jnp.concatenate([o0, o1], axis=1)
    h_out[...] = jnp.concatenate([o2, o3], axis=1)


def _make_final_call(interpret=False):
    return pl.pallas_call(
        _final_body,
        grid=(N // _BN_B,),
        in_specs=[
            pl.BlockSpec((NSLAB, _BN_B, DSLAB), lambda i: (0, i, 0)),
            pl.BlockSpec((NSLAB, _BN_B, DSLAB), lambda i: (0, i, 0)),
            pl.BlockSpec((_BN_B, 1), lambda i: (i, 0)),
        ],
        out_specs=[
            pl.BlockSpec((_BN_B, D_OUT), lambda i: (i, 0)),
            pl.BlockSpec((_BN_B, D_OUT), lambda i: (i, 0)),
        ],
        out_shape=[
            jax.ShapeDtypeStruct((N, D_OUT), jnp.float32),
            jax.ShapeDtypeStruct((N, D_OUT), jnp.float32),
        ],
        interpret=interpret,
    )


@jax.jit
def kernel(x, edge_index, W1, b1, W2, b2):
    deg_kernel, prop_kernel = _sc_kernels()
    src = edge_index[0].astype(jnp.int32)
    dst = edge_index[1].astype(jnp.int32)
    npad = EP - E
    padi = jnp.arange(npad, dtype=jnp.int32)
    src_pad = jnp.concatenate([src, N + 16 + (padi % 128)])
    dst_pad = jnp.concatenate([dst, N + 144 + (padi % 64)])
    dst3 = dst_pad.reshape(NTILE * NCORE, CHUNKS, 128)
    srcoff = (
        src_pad[None, :] + (jnp.arange(NSLAB, dtype=jnp.int32) * NP)[:, None]
    ).reshape(NSLAB * NTILE * NCORE, CHUNKS, 128)

    hist = deg_kernel(dst3)

    x_pad = jnp.zeros((NP, D_IN), jnp.float32).at[:N].set(x)
    w_cat = jnp.concatenate([W1.T, W2.T], axis=1)
    b_cat = jnp.concatenate([b1, b2])[None, :]

    g4, dinv = _make_fwd_call()(x_pad, w_cat, b_cat, hist[0], hist[1])

    s_flat = prop_kernel(g4.reshape(NSLAB * NP, DSLAB), srcoff, dst3)
    s4 = s_flat.reshape(NSLAB, NP, DSLAB)

    x_out, h_out = _make_final_call()(s4, g4, dinv)
    return (h_out, x_out)
